# baseline (speedup 1.0000x reference)
import functools

import jax
import jax.numpy as jnp
from jax import lax
from jax.experimental import pallas as pl
from jax.experimental.pallas import tpu as pltpu
from jax.experimental.pallas import tpu_sc as plsc

E = 320000
D = 128
DW = D // 2
L = 16
NUM_WORKERS = 32
E_PER_W = E // NUM_WORKERS
C = 80
NCHUNK = E_PER_W // C
G = C // L
PADW = L + 1

_mesh = plsc.VectorSubcoreMesh(core_axis_name="c", subcore_axis_name="s")


@functools.partial(
    pl.kernel,
    mesh=_mesh,
    out_type=jax.ShapeDtypeStruct((E,), jnp.float32),
    compiler_params=pltpu.CompilerParams(
        needs_layout_passes=False, use_tc_tiling_on_sc=False),
    scratch_types=[
        pltpu.VMEM((E_PER_W,), jnp.int32),
        pltpu.VMEM((E_PER_W,), jnp.int32),
        pltpu.VMEM((2, C, DW), jnp.int32),
        pltpu.VMEM((2, C, DW), jnp.int32),
        pltpu.VMEM((E_PER_W,), jnp.float32),
        pltpu.VMEM((C * PADW,), jnp.float32),
        pltpu.SemaphoreType.DMA,
        pltpu.SemaphoreType.DMA,
    ],
)
def _decode(z_hbm, src_hbm, dst_hbm, out_hbm,
            src_idx, dst_idx, srows, drows, outv, pad, sem_s, sem_d):
    wid = lax.axis_index("s") * 2 + lax.axis_index("c")
    base = wid * E_PER_W

    pltpu.sync_copy(src_hbm.at[pl.ds(base, E_PER_W)], src_idx)
    pltpu.sync_copy(dst_hbm.at[pl.ds(base, E_PER_W)], dst_idx)

    def issue(c, slot):
        off = c * C
        pltpu.async_copy(z_hbm.at[src_idx.at[pl.ds(off, C)]], srows.at[slot], sem_s)
        pltpu.async_copy(z_hbm.at[dst_idx.at[pl.ds(off, C)]], drows.at[slot], sem_d)

    def drain(c, slot):
        off = c * C
        pltpu.make_async_copy(
            z_hbm.at[src_idx.at[pl.ds(off, C)]], srows.at[slot], sem_s).wait()
        pltpu.make_async_copy(
            z_hbm.at[dst_idx.at[pl.ds(off, C)]], drows.at[slot], sem_d).wait()

    lanes = lax.iota(jnp.int32, L)

    def compute(c, slot):
        off = c * C
        sr = srows.at[slot]
        dr = drows.at[slot]

        def edge_body(e, carry):
            acc = jnp.zeros((L,), jnp.float32)
            for k in range(DW // L):
                sbf = plsc.bitcast(sr[e, pl.ds(k * L, L)], jnp.bfloat16)
                dbf = plsc.bitcast(dr[e, pl.ds(k * L, L)], jnp.bfloat16)
                sa, sb = plsc.unpack(sbf, format=plsc.PackFormat.INTERLEAVED)
                da, db = plsc.unpack(dbf, format=plsc.PackFormat.INTERLEAVED)
                acc = acc + sa * da
                acc = acc + sb * db
            pad[pl.ds(e * PADW, L)] = acc
            return carry

        lax.fori_loop(0, C, edge_body, 0, unroll=2)

        def group_body(g, carry):
            rows = (g * L + lanes) * PADW
            dots = plsc.load_gather(pad, [rows])
            for j in range(1, L):
                dots = dots + plsc.load_gather(pad, [rows + j])
            outv[pl.ds(off + g * L, L)] = 1.0 / (1.0 + jnp.exp(-dots))
            return carry

        lax.fori_loop(0, G, group_body, 0)

    issue(0, 0)
    issue(1, 1)

    def step(s, carry):
        c0 = 2 * s
        drain(c0, 0)
        compute(c0, 0)
        issue(c0 + 2, 0)
        drain(c0 + 1, 1)
        compute(c0 + 1, 1)

        @pl.when(s < (NCHUNK - 3) // 2)
        def _():
            issue(c0 + 3, 1)

        return carry

    lax.fori_loop(0, (NCHUNK - 1) // 2, step, 0)
    drain(NCHUNK - 1, 0)
    compute(NCHUNK - 1, 0)

    pltpu.sync_copy(outv, out_hbm.at[pl.ds(base, E_PER_W)])


def kernel(z, edge_index):
    zp = jax.lax.bitcast_convert_type(
        z.astype(jnp.bfloat16).reshape(z.shape[0], DW, 2), jnp.int32)
    idx = edge_index.astype(jnp.int32)
    return _decode(zp, idx[0], idx[1])


# speedup vs baseline: 1.1195x; 1.1195x over previous
import functools

import jax
import jax.numpy as jnp
from jax import lax
from jax.experimental import pallas as pl
from jax.experimental.pallas import tpu as pltpu
from jax.experimental.pallas import tpu_sc as plsc

E = 320000
D = 128
DW = D // 2
L = 16
NUM_WORKERS = 32
E_PER_W = E // NUM_WORKERS
C = 80
NCHUNK = E_PER_W // C
G = C // L
PADW = L + 1

_mesh = plsc.VectorSubcoreMesh(core_axis_name="c", subcore_axis_name="s")


@functools.partial(
    pl.kernel,
    mesh=_mesh,
    out_type=jax.ShapeDtypeStruct((E,), jnp.float32),
    compiler_params=pltpu.CompilerParams(
        needs_layout_passes=False, disable_bounds_checks=True),
    scratch_types=[
        pltpu.VMEM((E_PER_W,), jnp.int32),
        pltpu.VMEM((E_PER_W,), jnp.int32),
        pltpu.VMEM((2, C, D), jnp.float32),
        pltpu.VMEM((2, C, D), jnp.float32),
        pltpu.VMEM((E_PER_W,), jnp.float32),
        pltpu.VMEM((C * PADW,), jnp.float32),
        pltpu.SemaphoreType.DMA,
        pltpu.SemaphoreType.DMA,
    ],
)
def _decode(z_hbm, src_hbm, dst_hbm, out_hbm,
            src_idx, dst_idx, srows, drows, outv, pad, sem_s, sem_d):
    wid = lax.axis_index("s") * 2 + lax.axis_index("c")
    base = wid * E_PER_W

    pltpu.sync_copy(src_hbm.at[pl.ds(base, E_PER_W)], src_idx)
    pltpu.sync_copy(dst_hbm.at[pl.ds(base, E_PER_W)], dst_idx)

    def issue(c, slot):
        off = c * C
        pltpu.async_copy(z_hbm.at[src_idx.at[pl.ds(off, C)]], srows.at[slot], sem_s)
        pltpu.async_copy(z_hbm.at[dst_idx.at[pl.ds(off, C)]], drows.at[slot], sem_d)

    def drain(c, slot):
        off = c * C
        pltpu.make_async_copy(
            z_hbm.at[src_idx.at[pl.ds(off, C)]], srows.at[slot], sem_s).wait()
        pltpu.make_async_copy(
            z_hbm.at[dst_idx.at[pl.ds(off, C)]], drows.at[slot], sem_d).wait()

    lanes = lax.iota(jnp.int32, L)

    def compute(c, slot):
        off = c * C
        sr = srows.at[slot]
        dr = drows.at[slot]

        def edge_body(e, carry):
            acc0 = sr[e, pl.ds(0, L)] * dr[e, pl.ds(0, L)]
            acc1 = sr[e, pl.ds(L, L)] * dr[e, pl.ds(L, L)]
            for k in range(2, D // L, 2):
                acc0 = acc0 + sr[e, pl.ds(k * L, L)] * dr[e, pl.ds(k * L, L)]
                acc1 = acc1 + sr[e, pl.ds((k + 1) * L, L)] * dr[e, pl.ds((k + 1) * L, L)]
            pad[pl.ds(e * PADW, L)] = acc0 + acc1
            return carry

        lax.fori_loop(0, C, edge_body, 0, unroll=2)

        def group_body(g, carry):
            rows = (g * L + lanes) * PADW
            dots = plsc.load_gather(pad, [rows])
            for j in range(1, L):
                dots = dots + plsc.load_gather(pad, [rows + j])
            outv[pl.ds(off + g * L, L)] = 1.0 / (1.0 + jnp.exp(-dots))
            return carry

        lax.fori_loop(0, G, group_body, 0)

    issue(0, 0)
    issue(1, 1)

    def step(s, carry):
        c0 = 2 * s
        drain(c0, 0)
        compute(c0, 0)
        issue(c0 + 2, 0)
        drain(c0 + 1, 1)
        compute(c0 + 1, 1)

        @pl.when(s < (NCHUNK - 3) // 2)
        def _():
            issue(c0 + 3, 1)

        return carry

    lax.fori_loop(0, (NCHUNK - 1) // 2, step, 0)
    drain(NCHUNK - 1, 0)
    compute(NCHUNK - 1, 0)

    pltpu.sync_copy(outv, out_hbm.at[pl.ds(base, E_PER_W)])


def kernel(z, edge_index):
    idx = edge_index.astype(jnp.int32)
    return _decode(z, idx[0], idx[1])


# speedup vs baseline: 1.3103x; 1.1704x over previous
import functools

import jax
import jax.numpy as jnp
from jax import lax
from jax.experimental import pallas as pl
from jax.experimental.pallas import tpu as pltpu
from jax.experimental.pallas import tpu_sc as plsc

E = 320000
D = 128
L = 16
NUM_WORKERS = 32
E_PER_W = E // NUM_WORKERS
C = 80
NCHUNK = E_PER_W // C
G = C // L

_mesh = plsc.VectorSubcoreMesh(core_axis_name="c", subcore_axis_name="s")


@functools.partial(
    pl.kernel,
    mesh=_mesh,
    out_type=jax.ShapeDtypeStruct((E,), jnp.float32),
    compiler_params=pltpu.CompilerParams(
        needs_layout_passes=False, disable_bounds_checks=True),
    scratch_types=[
        pltpu.VMEM((E_PER_W,), jnp.int32),
        pltpu.VMEM((E_PER_W,), jnp.int32),
        pltpu.VMEM((2, C, D), jnp.float32),
        pltpu.VMEM((2, C, D), jnp.float32),
        pltpu.VMEM((E_PER_W,), jnp.float32),
        pltpu.SemaphoreType.DMA,
        pltpu.SemaphoreType.DMA,
    ],
)
def _decode(z_hbm, src_hbm, dst_hbm, out_hbm,
            src_idx, dst_idx, srows, drows, outv, sem_s, sem_d):
    wid = lax.axis_index("s") * 2 + lax.axis_index("c")
    base = wid * E_PER_W

    pltpu.sync_copy(src_hbm.at[pl.ds(base, E_PER_W)], src_idx)
    pltpu.sync_copy(dst_hbm.at[pl.ds(base, E_PER_W)], dst_idx)

    def issue(c, slot):
        off = c * C
        pltpu.async_copy(z_hbm.at[src_idx.at[pl.ds(off, C)]], srows.at[slot], sem_s)
        pltpu.async_copy(z_hbm.at[dst_idx.at[pl.ds(off, C)]], drows.at[slot], sem_d)

    def drain(c, slot):
        off = c * C
        pltpu.make_async_copy(
            z_hbm.at[src_idx.at[pl.ds(off, C)]], srows.at[slot], sem_s).wait()
        pltpu.make_async_copy(
            z_hbm.at[dst_idx.at[pl.ds(off, C)]], drows.at[slot], sem_d).wait()

    lanes = lax.iota(jnp.int32, L)
    lanes_d = lanes * D
    zv = jnp.zeros((L,), jnp.int32)
    colvs = [(lanes + b) & (L - 1) for b in range(L)]

    def compute(c, slot):
        off = c * C
        sr = srows.at[slot]
        dr = drows.at[slot]

        def group_body(g, carry):
            ridx = g * (L * D) + lanes_d

            def k_body(k, accs):
                a0, a1, a2, a3 = accs
                ridx_k = ridx + k * L
                for b in range(L):
                    idx = ridx_k + colvs[b]
                    sv = plsc.load_gather(sr, [zv, idx])
                    dv = plsc.load_gather(dr, [zv, idx])
                    prod = sv * dv
                    if b % 4 == 0:
                        a0 = a0 + prod
                    elif b % 4 == 1:
                        a1 = a1 + prod
                    elif b % 4 == 2:
                        a2 = a2 + prod
                    else:
                        a3 = a3 + prod
                return a0, a1, a2, a3

            zf = jnp.zeros((L,), jnp.float32)
            a0, a1, a2, a3 = lax.fori_loop(0, D // L, k_body, (zf, zf, zf, zf))
            dots = (a0 + a1) + (a2 + a3)
            outv[pl.ds(off + g * L, L)] = 1.0 / (1.0 + jnp.exp(-dots))
            return carry

        lax.fori_loop(0, G, group_body, 0)

    issue(0, 0)
    issue(1, 1)

    def step(s, carry):
        c0 = 2 * s
        drain(c0, 0)
        compute(c0, 0)
        issue(c0 + 2, 0)
        drain(c0 + 1, 1)
        compute(c0 + 1, 1)

        @pl.when(s < (NCHUNK - 3) // 2)
        def _():
            issue(c0 + 3, 1)

        return carry

    lax.fori_loop(0, (NCHUNK - 1) // 2, step, 0)
    drain(NCHUNK - 1, 0)
    compute(NCHUNK - 1, 0)

    pltpu.sync_copy(outv, out_hbm.at[pl.ds(base, E_PER_W)])


def kernel(z, edge_index):
    idx = edge_index.astype(jnp.int32)
    return _decode(z, idx[0], idx[1])


# speedup vs baseline: 1.4655x; 1.1185x over previous
import functools

import jax
import jax.numpy as jnp
from jax import lax
from jax.experimental import pallas as pl
from jax.experimental.pallas import tpu as pltpu
from jax.experimental.pallas import tpu_sc as plsc

E = 320000
D = 128
DW = D // 2
L = 16
NUM_WORKERS = 32
E_PER_W = E // NUM_WORKERS
C = 80
NCHUNK = E_PER_W // C
G = C // L

_mesh = plsc.VectorSubcoreMesh(core_axis_name="c", subcore_axis_name="s")


@functools.partial(
    pl.kernel,
    mesh=_mesh,
    out_type=jax.ShapeDtypeStruct((E,), jnp.float32),
    compiler_params=pltpu.CompilerParams(
        needs_layout_passes=False, disable_bounds_checks=True,
        use_tc_tiling_on_sc=False),
    scratch_types=[
        pltpu.VMEM((E_PER_W,), jnp.int32),
        pltpu.VMEM((E_PER_W,), jnp.int32),
        pltpu.VMEM((2, C, DW), jnp.int32),
        pltpu.VMEM((2, C, DW), jnp.int32),
        pltpu.VMEM((E_PER_W,), jnp.float32),
        pltpu.SemaphoreType.DMA,
        pltpu.SemaphoreType.DMA,
    ],
)
def _decode(z_hbm, src_hbm, dst_hbm, out_hbm,
            src_idx, dst_idx, srows, drows, outv, sem_s, sem_d):
    wid = lax.axis_index("s") * 2 + lax.axis_index("c")
    base = wid * E_PER_W

    pltpu.sync_copy(src_hbm.at[pl.ds(base, E_PER_W)], src_idx)
    pltpu.sync_copy(dst_hbm.at[pl.ds(base, E_PER_W)], dst_idx)

    def issue(c, slot):
        off = c * C
        pltpu.async_copy(z_hbm.at[src_idx.at[pl.ds(off, C)]], srows.at[slot], sem_s)
        pltpu.async_copy(z_hbm.at[dst_idx.at[pl.ds(off, C)]], drows.at[slot], sem_d)

    def drain(c, slot):
        off = c * C
        pltpu.make_async_copy(
            z_hbm.at[src_idx.at[pl.ds(off, C)]], srows.at[slot], sem_s).wait()
        pltpu.make_async_copy(
            z_hbm.at[dst_idx.at[pl.ds(off, C)]], drows.at[slot], sem_d).wait()

    lanes = lax.iota(jnp.int32, L)
    lanes_dw = lanes * DW
    zv = jnp.zeros((L,), jnp.int32)
    colvs = [(lanes + b) & (L - 1) for b in range(L)]

    def compute(c, slot):
        off = c * C
        sr = srows.at[slot]
        dr = drows.at[slot]

        def group_body(g, carry):
            ridx = g * (L * DW) + lanes_dw

            def k_body(k, accs):
                a0, a1, a2, a3 = accs
                ridx_k = ridx + k * L
                for b in range(L):
                    idx = ridx_k + colvs[b]
                    sw = plsc.load_gather(sr, [zv, idx])
                    dw = plsc.load_gather(dr, [zv, idx])
                    prod = (plsc.bitcast(sw, jnp.bfloat16)
                            * plsc.bitcast(dw, jnp.bfloat16))
                    pa, pb = plsc.unpack(prod, format=plsc.PackFormat.INTERLEAVED)
                    if b % 2 == 0:
                        a0 = a0 + pa
                        a1 = a1 + pb
                    else:
                        a2 = a2 + pa
                        a3 = a3 + pb
                return a0, a1, a2, a3

            zf = jnp.zeros((L,), jnp.float32)
            a0, a1, a2, a3 = lax.fori_loop(0, DW // L, k_body, (zf, zf, zf, zf))
            dots = (a0 + a1) + (a2 + a3)
            outv[pl.ds(off + g * L, L)] = 1.0 / (1.0 + jnp.exp(-dots))
            return carry

        lax.fori_loop(0, G, group_body, 0)

    issue(0, 0)
    issue(1, 1)

    def step(s, carry):
        c0 = 2 * s
        drain(c0, 0)
        compute(c0, 0)
        issue(c0 + 2, 0)
        drain(c0 + 1, 1)
        compute(c0 + 1, 1)

        @pl.when(s < (NCHUNK - 3) // 2)
        def _():
            issue(c0 + 3, 1)

        return carry

    lax.fori_loop(0, (NCHUNK - 1) // 2, step, 0)
    drain(NCHUNK - 1, 0)
    compute(NCHUNK - 1, 0)

    pltpu.sync_copy(outv, out_hbm.at[pl.ds(base, E_PER_W)])


def kernel(z, edge_index):
    zp = jax.lax.bitcast_convert_type(
        z.astype(jnp.bfloat16).reshape(z.shape[0], DW, 2), jnp.int32)
    idx = edge_index.astype(jnp.int32)
    return _decode(zp, idx[0], idx[1])


# speedup vs baseline: 1.8826x; 1.2846x over previous
import functools

import jax
import jax.numpy as jnp
from jax import lax
from jax.experimental import pallas as pl
from jax.experimental.pallas import tpu as pltpu
from jax.experimental.pallas import tpu_sc as plsc

E = 320000
D = 128
DW = D // 2
L = 16
NUM_WORKERS = 32
E_PER_W = E // NUM_WORKERS
C = 80
NCHUNK = E_PER_W // C
G = C // L

_mesh = plsc.VectorSubcoreMesh(core_axis_name="c", subcore_axis_name="s")


@functools.partial(
    pl.kernel,
    mesh=_mesh,
    out_type=jax.ShapeDtypeStruct((E,), jnp.float32),
    compiler_params=pltpu.CompilerParams(
        needs_layout_passes=False, disable_bounds_checks=True,
        use_tc_tiling_on_sc=False),
    scratch_types=[
        pltpu.VMEM((E_PER_W,), jnp.int32),
        pltpu.VMEM((E_PER_W,), jnp.int32),
        pltpu.VMEM((2, C, DW), jnp.int32),
        pltpu.VMEM((2, C, DW), jnp.int32),
        pltpu.VMEM((E_PER_W,), jnp.float32),
        pltpu.SemaphoreType.DMA,
        pltpu.SemaphoreType.DMA,
    ],
)
def _decode(z_hbm, ei_hbm, out_hbm,
            src_idx, dst_idx, srows, drows, outv, sem_s, sem_d):
    wid = lax.axis_index("s") * 2 + lax.axis_index("c")
    base = wid * E_PER_W

    pltpu.sync_copy(ei_hbm.at[0, pl.ds(base, E_PER_W)], src_idx)
    pltpu.sync_copy(ei_hbm.at[1, pl.ds(base, E_PER_W)], dst_idx)

    def issue(c, slot):
        off = c * C
        pltpu.async_copy(z_hbm.at[src_idx.at[pl.ds(off, C)]], srows.at[slot], sem_s)
        pltpu.async_copy(z_hbm.at[dst_idx.at[pl.ds(off, C)]], drows.at[slot], sem_d)

    def drain(c, slot):
        off = c * C
        pltpu.make_async_copy(
            z_hbm.at[src_idx.at[pl.ds(off, C)]], srows.at[slot], sem_s).wait()
        pltpu.make_async_copy(
            z_hbm.at[dst_idx.at[pl.ds(off, C)]], drows.at[slot], sem_d).wait()

    lanes = lax.iota(jnp.int32, L)
    lanes_dw = lanes * DW
    zv = jnp.zeros((L,), jnp.int32)
    colvs = [(lanes + b) & (L - 1) for b in range(L)]

    def compute(c, slot):
        off = c * C
        sr = srows.at[slot]
        dr = drows.at[slot]

        def group_body(g, carry):
            ridx = g * (L * DW) + lanes_dw

            def k_body(k, accs):
                a0, a1, a2, a3 = accs
                ridx_k = ridx + k * L
                for b in range(L):
                    idx = ridx_k + colvs[b]
                    sw = plsc.load_gather(sr, [zv, idx])
                    dw = plsc.load_gather(dr, [zv, idx])
                    prod = (plsc.bitcast(sw, jnp.bfloat16)
                            * plsc.bitcast(dw, jnp.bfloat16))
                    pa, pb = plsc.unpack(prod, format=plsc.PackFormat.INTERLEAVED)
                    if b % 2 == 0:
                        a0 = a0 + pa
                        a1 = a1 + pb
                    else:
                        a2 = a2 + pa
                        a3 = a3 + pb
                return a0, a1, a2, a3

            zf = jnp.zeros((L,), jnp.float32)
            a0, a1, a2, a3 = lax.fori_loop(0, DW // L, k_body, (zf, zf, zf, zf))
            dots = (a0 + a1) + (a2 + a3)
            outv[pl.ds(off + g * L, L)] = 1.0 / (1.0 + jnp.exp(-dots))
            return carry

        lax.fori_loop(0, G, group_body, 0)

    issue(0, 0)
    issue(1, 1)

    def step(s, carry):
        c0 = 2 * s
        drain(c0, 0)
        compute(c0, 0)
        issue(c0 + 2, 0)
        drain(c0 + 1, 1)
        compute(c0 + 1, 1)

        @pl.when(s < (NCHUNK - 3) // 2)
        def _():
            issue(c0 + 3, 1)

        return carry

    lax.fori_loop(0, (NCHUNK - 1) // 2, step, 0)
    drain(NCHUNK - 1, 0)
    compute(NCHUNK - 1, 0)

    pltpu.sync_copy(outv, out_hbm.at[pl.ds(base, E_PER_W)])


def kernel(z, edge_index):
    u = jax.lax.bitcast_convert_type(z, jnp.uint32)
    r = (u + jnp.uint32(0x7FFF) + ((u >> 16) & jnp.uint32(1))) >> 16
    zp = jax.lax.bitcast_convert_type(
        (r[:, DW:] << 16) | r[:, :DW], jnp.int32)
    return _decode(zp, edge_index.astype(jnp.int32))


# speedup vs baseline: 1.8990x; 1.0087x over previous
import functools

import jax
import jax.numpy as jnp
from jax import lax
from jax.experimental import pallas as pl
from jax.experimental.pallas import tpu as pltpu
from jax.experimental.pallas import tpu_sc as plsc

E = 320000
D = 128
DW = D // 2
L = 16
NUM_WORKERS = 32
E_PER_W = E // NUM_WORKERS
C = 80
NCHUNK = E_PER_W // C
G = C // L

_mesh = plsc.VectorSubcoreMesh(core_axis_name="c", subcore_axis_name="s")


@functools.partial(
    pl.kernel,
    mesh=_mesh,
    out_type=jax.ShapeDtypeStruct((E,), jnp.float32),
    compiler_params=pltpu.CompilerParams(
        needs_layout_passes=False, disable_bounds_checks=True,
        use_tc_tiling_on_sc=False),
    scratch_types=[
        pltpu.VMEM((E_PER_W,), jnp.int32),
        pltpu.VMEM((E_PER_W,), jnp.int32),
        pltpu.VMEM((2, C, DW), jnp.int32),
        pltpu.VMEM((2, C, DW), jnp.int32),
        pltpu.VMEM((E_PER_W,), jnp.float32),
        pltpu.SemaphoreType.DMA,
        pltpu.SemaphoreType.DMA,
    ],
)
def _decode(z_hbm, ei_hbm, out_hbm,
            src_idx, dst_idx, srows, drows, outv, sem_s, sem_d):
    wid = lax.axis_index("s") * 2 + lax.axis_index("c")
    base = wid * E_PER_W

    pltpu.sync_copy(ei_hbm.at[0, pl.ds(base, E_PER_W)], src_idx)
    pltpu.sync_copy(ei_hbm.at[1, pl.ds(base, E_PER_W)], dst_idx)

    def issue(c, slot):
        off = c * C
        pltpu.async_copy(z_hbm.at[src_idx.at[pl.ds(off, C)]], srows.at[slot], sem_s)
        pltpu.async_copy(z_hbm.at[dst_idx.at[pl.ds(off, C)]], drows.at[slot], sem_d)

    def drain(c, slot):
        off = c * C
        pltpu.make_async_copy(
            z_hbm.at[src_idx.at[pl.ds(off, C)]], srows.at[slot], sem_s).wait()
        pltpu.make_async_copy(
            z_hbm.at[dst_idx.at[pl.ds(off, C)]], drows.at[slot], sem_d).wait()

    lanes = lax.iota(jnp.int32, L)
    lanes_dw = lanes * DW
    zv = jnp.zeros((L,), jnp.int32)
    colvs = [(lanes + b) & (L - 1) for b in range(L)]

    def compute(c, slot):
        off = c * C
        sr = srows.at[slot]
        dr = drows.at[slot]

        def group_body(g, carry):
            ridx = g * (L * DW) + lanes_dw

            def k_body(k, accs):
                a0, a1, a2, a3 = accs
                ridx_k = ridx + k * L
                for b in range(L):
                    idx = ridx_k + colvs[b]
                    sw = plsc.load_gather(sr, [zv, idx])
                    dw = plsc.load_gather(dr, [zv, idx])
                    prod = (plsc.bitcast(sw, jnp.bfloat16)
                            * plsc.bitcast(dw, jnp.bfloat16))
                    pa, pb = plsc.unpack(prod, format=plsc.PackFormat.INTERLEAVED)
                    if b % 2 == 0:
                        a0 = a0 + pa
                        a1 = a1 + pb
                    else:
                        a2 = a2 + pa
                        a3 = a3 + pb
                return a0, a1, a2, a3

            zf = jnp.zeros((L,), jnp.float32)
            a0, a1, a2, a3 = lax.fori_loop(0, DW // L, k_body, (zf, zf, zf, zf))
            outv[pl.ds(off + g * L, L)] = (a0 + a1) + (a2 + a3)
            return carry

        lax.fori_loop(0, G, group_body, 0)
        for g in range(G):
            v = outv[pl.ds(off + g * L, L)]
            outv[pl.ds(off + g * L, L)] = 1.0 / (1.0 + jnp.exp(-v))

    issue(0, 0)
    issue(1, 1)

    def step(s, carry):
        c0 = 2 * s
        drain(c0, 0)
        compute(c0, 0)
        issue(c0 + 2, 0)
        drain(c0 + 1, 1)
        compute(c0 + 1, 1)

        @pl.when(s < (NCHUNK - 3) // 2)
        def _():
            issue(c0 + 3, 1)

        return carry

    lax.fori_loop(0, (NCHUNK - 1) // 2, step, 0)
    drain(NCHUNK - 1, 0)
    compute(NCHUNK - 1, 0)

    pltpu.sync_copy(outv, out_hbm.at[pl.ds(base, E_PER_W)])


def kernel(z, edge_index):
    u = jax.lax.bitcast_convert_type(z, jnp.uint32)
    r = (u + jnp.uint32(0x7FFF) + ((u >> 16) & jnp.uint32(1))) >> 16
    zp = jax.lax.bitcast_convert_type(
        (r[:, DW:] << 16) | r[:, :DW], jnp.int32)
    return _decode(zp, edge_index.astype(jnp.int32))


# speedup vs baseline: 2.3479x; 1.2364x over previous
import functools

import jax
import jax.numpy as jnp
from jax import lax
from jax.experimental import pallas as pl
from jax.experimental.pallas import tpu as pltpu
from jax.experimental.pallas import tpu_sc as plsc

E = 320000
D = 128
DW = D // 2
L = 16
NUM_WORKERS = 32
E_PER_W = E // NUM_WORKERS
C = 400
NCHUNK = E_PER_W // C
G = C // L

_mesh = plsc.VectorSubcoreMesh(core_axis_name="c", subcore_axis_name="s")


@functools.partial(
    pl.kernel,
    mesh=_mesh,
    out_type=jax.ShapeDtypeStruct((E,), jnp.float32),
    compiler_params=pltpu.CompilerParams(
        needs_layout_passes=False, disable_bounds_checks=True,
        use_tc_tiling_on_sc=False),
    scratch_types=[
        pltpu.VMEM((E_PER_W,), jnp.int32),
        pltpu.VMEM((E_PER_W,), jnp.int32),
        pltpu.VMEM((2, C, DW), jnp.int32),
        pltpu.VMEM((2, C, DW), jnp.int32),
        pltpu.VMEM((2, C), jnp.float32),
        pltpu.SemaphoreType.DMA,
        pltpu.SemaphoreType.DMA,
        pltpu.SemaphoreType.DMA,
    ],
)
def _decode(z_hbm, ei_hbm, out_hbm,
            src_idx, dst_idx, srows, drows, outv, sem_s, sem_d, sem_o):
    wid = lax.axis_index("s") * 2 + lax.axis_index("c")
    base = wid * E_PER_W

    pltpu.sync_copy(ei_hbm.at[0, pl.ds(base, E_PER_W)], src_idx)
    pltpu.sync_copy(ei_hbm.at[1, pl.ds(base, E_PER_W)], dst_idx)

    def issue(c, slot):
        off = c * C
        pltpu.async_copy(z_hbm.at[src_idx.at[pl.ds(off, C)]], srows.at[slot], sem_s)
        pltpu.async_copy(z_hbm.at[dst_idx.at[pl.ds(off, C)]], drows.at[slot], sem_d)

    def drain(c, slot):
        off = c * C
        pltpu.make_async_copy(
            z_hbm.at[src_idx.at[pl.ds(off, C)]], srows.at[slot], sem_s).wait()
        pltpu.make_async_copy(
            z_hbm.at[dst_idx.at[pl.ds(off, C)]], drows.at[slot], sem_d).wait()

    lanes = lax.iota(jnp.int32, L)
    lanes_dw = lanes * DW
    zv = jnp.zeros((L,), jnp.int32)
    colvs = [(lanes + b) & (L - 1) for b in range(L)]

    def store_out(c, slot):
        pltpu.async_copy(outv.at[slot], out_hbm.at[pl.ds(base + c * C, C)], sem_o)

    def drain_out(c, slot):
        pltpu.make_async_copy(
            outv.at[slot], out_hbm.at[pl.ds(base + c * C, C)], sem_o).wait()

    def compute(c, slot):
        sr = srows.at[slot]
        dr = drows.at[slot]

        def group_body(g, carry):
            ridx = g * (L * DW) + lanes_dw

            def k_body(k, accs):
                a0, a1, a2, a3 = accs
                ridx_k = ridx + k * L
                for b in range(L):
                    idx = ridx_k + colvs[b]
                    sw = plsc.load_gather(sr, [zv, idx])
                    dw = plsc.load_gather(dr, [zv, idx])
                    prod = (plsc.bitcast(sw, jnp.bfloat16)
                            * plsc.bitcast(dw, jnp.bfloat16))
                    pa, pb = plsc.unpack(prod, format=plsc.PackFormat.INTERLEAVED)
                    if b % 2 == 0:
                        a0 = a0 + pa
                        a1 = a1 + pb
                    else:
                        a2 = a2 + pa
                        a3 = a3 + pb
                return a0, a1, a2, a3

            zf = jnp.zeros((L,), jnp.float32)
            a0, a1, a2, a3 = lax.fori_loop(0, DW // L, k_body, (zf, zf, zf, zf))
            outv[slot, pl.ds(g * L, L)] = (a0 + a1) + (a2 + a3)
            return carry

        lax.fori_loop(0, G, group_body, 0)

        def sig_body(g, carry):
            v = outv[slot, pl.ds(g * L, L)]
            outv[slot, pl.ds(g * L, L)] = 1.0 / (1.0 + jnp.exp(-v))
            return carry

        lax.fori_loop(0, G, sig_body, 0, unroll=5)

    issue(0, 0)
    issue(1, 1)

    def step(s, carry):
        c0 = 2 * s
        drain(c0, 0)

        @pl.when(s > 0)
        def _():
            drain_out(c0 - 2, 0)

        compute(c0, 0)
        store_out(c0, 0)
        issue(c0 + 2, 0)
        drain(c0 + 1, 1)

        @pl.when(s > 0)
        def _():
            drain_out(c0 - 1, 1)

        compute(c0 + 1, 1)
        store_out(c0 + 1, 1)

        @pl.when(s < (NCHUNK - 3) // 2)
        def _():
            issue(c0 + 3, 1)

        return carry

    lax.fori_loop(0, (NCHUNK - 1) // 2, step, 0)
    drain(NCHUNK - 1, 0)
    drain_out(NCHUNK - 3, 0)
    compute(NCHUNK - 1, 0)
    store_out(NCHUNK - 1, 0)
    drain_out(NCHUNK - 2, 1)
    drain_out(NCHUNK - 1, 0)


def kernel(z, edge_index):
    u = jax.lax.bitcast_convert_type(z, jnp.uint32)
    r = (u + jnp.uint32(0x7FFF) + ((u >> 16) & jnp.uint32(1))) >> 16
    zp = jax.lax.bitcast_convert_type(
        (r[:, DW:] << 16) | r[:, :DW], jnp.int32)
    return _decode(zp, edge_index.astype(jnp.int32))
